# Initial kernel scaffold; baseline (speedup 1.0000x reference)
import jax
import jax.numpy as jnp
from jax.experimental import pallas as pl


def kernel(candidates, table, natural_contribs):
    raise NotImplementedError("write your pallas kernel here")


# speedup vs baseline: 1.7196x; 1.7196x over previous
import functools

import jax
import jax.numpy as jnp
from jax import lax
from jax.experimental import pallas as pl
from jax.experimental.pallas import tpu as pltpu
from jax.experimental.pallas import tpu_sc as plsc

ALPHA = 20
KMER = 5
MOTIF_EFFECT = 2.0
NC, NS, LANES = 2, 16, 16
NW = NC * NS
DMA_CHUNK = 128
DEPTH = 8


def _body(B, Lseq, P, cand_hbm, table_hbm, ncb_hbm, out_hbm,
          slab, ncb_v, idx_v, val_v, out_v, sem):
    W = Lseq - KMER + 1
    RPW = B // NW
    NCH = RPW // LANES
    FL = W * RPW
    NJ = FL // DMA_CHUNK
    NG = NJ // DEPTH
    POW_TOP = ALPHA ** (KMER - 1)

    cid = lax.axis_index("c")
    sid = lax.axis_index("s")
    wid = sid * NC + cid
    base = wid * RPW

    pltpu.sync_copy(cand_hbm.at[pl.ds(base * Lseq, RPW * Lseq)], slab)
    pltpu.sync_copy(ncb_hbm, ncb_v)

    def idx_chunk(c, carry):
        rows = (c * LANES + lax.iota(jnp.int32, LANES)) * Lseq

        def col(j):
            return plsc.load_gather(slab, [rows + j])

        h = col(0)
        for j in range(1, KMER):
            h = h * ALPHA + col(j)
        idx_v[pl.ds(c * LANES, LANES)] = h
        for w in range(1, W):
            h = (h - col(w - 1) * POW_TOP) * ALPHA + col(w + KMER - 1)
            idx_v[pl.ds(w * RPW + c * LANES, LANES)] = h
        return carry

    lax.fori_loop(0, NCH, idx_chunk, None)

    def start_group(g):
        for b in range(DEPTH):
            j = g * DEPTH + b
            pltpu.make_async_copy(
                table_hbm.at[idx_v.at[pl.ds(j * DMA_CHUNK, DMA_CHUNK)]],
                val_v.at[pl.ds(j * DMA_CHUNK, DMA_CHUNK)],
                sem,
            ).start()

    def wait_group(g):
        pltpu.make_async_copy(
            table_hbm.at[pl.ds(0, DEPTH * DMA_CHUNK)],
            val_v.at[pl.ds(g * DEPTH * DMA_CHUNK, DEPTH * DMA_CHUNK)],
            sem,
        ).wait()

    start_group(0)

    def gather_loop(g, carry):
        start_group(g)
        wait_group(g - 1)
        return carry

    lax.fori_loop(1, NG, gather_loop, None)
    wait_group(NG - 1)

    inv_p = 1.0 / P

    def red_chunk(c, carry):
        cnt = val_v[pl.ds(c * LANES, LANES)]
        for w in range(1, W):
            cnt = cnt + val_v[pl.ds(w * RPW + c * LANES, LANES)]
        contrib = jnp.minimum(cnt, 1.0) * MOTIF_EFFECT
        s = jnp.zeros((LANES,), jnp.float32)
        for p in range(P):
            x = contrib + ncb_v[p, :]
            s = s + 1.0 / (1.0 + jnp.exp(-x))
        out_v[pl.ds(c * LANES, LANES)] = s * inv_p
        return carry

    lax.fori_loop(0, NCH, red_chunk, None)
    pltpu.sync_copy(out_v, out_hbm.at[pl.ds(base, RPW)])


@jax.jit
def kernel(candidates, table, natural_contribs):
    B, Lseq = candidates.shape
    P = natural_contribs.shape[0]
    W = Lseq - KMER + 1
    RPW = B // NW
    FL = W * RPW

    cand = candidates.astype(jnp.int32).reshape(-1)
    table_flat = table.reshape(-1).astype(jnp.float32)
    ncb = jnp.broadcast_to(
        natural_contribs.astype(jnp.float32)[:, None], (P, LANES))

    mesh = plsc.VectorSubcoreMesh(
        core_axis_name="c", subcore_axis_name="s",
        num_cores=NC, num_subcores=NS)

    run = pl.kernel(
        functools.partial(_body, B, Lseq, P),
        out_type=jax.ShapeDtypeStruct((B,), jnp.float32),
        mesh=mesh,
        compiler_params=pltpu.CompilerParams(needs_layout_passes=False),
        scratch_types=[
            pltpu.VMEM((RPW * Lseq,), jnp.int32),
            pltpu.VMEM((P, LANES), jnp.float32),
            pltpu.VMEM((FL,), jnp.int32),
            pltpu.VMEM((FL,), jnp.float32),
            pltpu.VMEM((RPW,), jnp.float32),
            pltpu.SemaphoreType.DMA,
        ],
    )
    return run(cand, table_flat, ncb)


# speedup vs baseline: 2.1258x; 1.2362x over previous
import functools

import jax
import jax.numpy as jnp
from jax import lax
from jax.experimental import pallas as pl
from jax.experimental.pallas import tpu as pltpu
from jax.experimental.pallas import tpu_sc as plsc

ALPHA = 20
KMER = 5
MOTIF_EFFECT = 2.0
NC, NS, LANES = 2, 16, 16
NW = NC * NS
GROUPS = 4
GROW = 128


def _body(B, Lseq, P, cand_hbm, table_hbm, ncb_hbm, out_hbm,
          slab, ncb_v, idx_v, val_v, out_v, sem):
    W = Lseq - KMER + 1
    RPW = B // NW
    GCH = GROW // LANES
    POW_TOP = ALPHA ** (KMER - 1)

    cid = lax.axis_index("c")
    sid = lax.axis_index("s")
    wid = sid * NC + cid
    base = wid * RPW

    pltpu.sync_copy(cand_hbm.at[pl.ds(base * Lseq, RPW * Lseq)], slab)
    pltpu.sync_copy(ncb_hbm, ncb_v)

    def compute_idx(g):
        def idx_chunk(c, carry):
            rows = ((g * GROW + c * LANES) + lax.iota(jnp.int32, LANES)) * Lseq
            cols = {}

            def col(j):
                if j not in cols:
                    cols[j] = plsc.load_gather(slab, [rows + j])
                return cols[j]

            h = col(0)
            for j in range(1, KMER):
                h = h * ALPHA + col(j)
            idx_v[pl.ds(g * W * GROW + c * LANES, LANES)] = h
            for w in range(1, W):
                h = (h - col(w - 1) * POW_TOP) * ALPHA + col(w + KMER - 1)
                idx_v[pl.ds(g * W * GROW + w * GROW + c * LANES, LANES)] = h
            return carry

        lax.fori_loop(0, GCH, idx_chunk, None)

    def gather_dma(g):
        return pltpu.make_async_copy(
            table_hbm.at[idx_v.at[pl.ds(g * W * GROW, W * GROW)]],
            val_v.at[pl.ds(g * W * GROW, W * GROW)], sem.at[g])

    inv_p = 1.0 / P

    def reduce_group(g):
        def red_chunk(c, carry):
            cnt = val_v[pl.ds(g * W * GROW + c * LANES, LANES)]
            for w in range(1, W):
                cnt = cnt + val_v[pl.ds(g * W * GROW + w * GROW + c * LANES, LANES)]
            contrib = jnp.minimum(cnt, 1.0) * MOTIF_EFFECT
            s = jnp.zeros((LANES,), jnp.float32)
            for p in range(P):
                x = contrib + ncb_v[p, :]
                s = s + 1.0 / (1.0 + jnp.exp(-x))
            out_v[pl.ds(g * GROW + c * LANES, LANES)] = s * inv_p
            return carry

        lax.fori_loop(0, GCH, red_chunk, None)

    compute_idx(0)
    gather_dma(0).start()
    for g in range(1, GROUPS):
        compute_idx(g)
        gather_dma(g).start()
    for g in range(GROUPS):
        gather_dma(g).wait()
        reduce_group(g)

    pltpu.sync_copy(out_v, out_hbm.at[pl.ds(base, RPW)])


@jax.jit
def kernel(candidates, table, natural_contribs):
    B, Lseq = candidates.shape
    P = natural_contribs.shape[0]
    W = Lseq - KMER + 1
    RPW = B // NW

    cand = candidates.astype(jnp.int32).reshape(-1)
    table_flat = table.reshape(-1).astype(jnp.float32)
    ncb = jnp.broadcast_to(
        natural_contribs.astype(jnp.float32)[:, None], (P, LANES))

    mesh = plsc.VectorSubcoreMesh(
        core_axis_name="c", subcore_axis_name="s",
        num_cores=NC, num_subcores=NS)

    run = pl.kernel(
        functools.partial(_body, B, Lseq, P),
        out_type=jax.ShapeDtypeStruct((B,), jnp.float32),
        mesh=mesh,
        compiler_params=pltpu.CompilerParams(needs_layout_passes=False),
        scratch_types=[
            pltpu.VMEM((RPW * Lseq,), jnp.int32),
            pltpu.VMEM((P, LANES), jnp.float32),
            pltpu.VMEM((GROUPS * W * GROW,), jnp.int32),
            pltpu.VMEM((GROUPS * W * GROW,), jnp.float32),
            pltpu.VMEM((RPW,), jnp.float32),
            pltpu.SemaphoreType.DMA((GROUPS,)),
        ],
    )
    return run(cand, table_flat, ncb)


# speedup vs baseline: 2.3122x; 1.0877x over previous
import functools

import jax
import jax.numpy as jnp
from jax import lax
from jax.experimental import pallas as pl
from jax.experimental.pallas import tpu as pltpu
from jax.experimental.pallas import tpu_sc as plsc

ALPHA = 20
KMER = 5
MOTIF_EFFECT = 2.0
NC, NS, LANES = 2, 16, 16
NW = NC * NS
GROUPS = 4
GROW = 128


def _body(B, Lseq, P, cand_hbm, table_hbm, ncb_hbm, out_hbm,
          slab, ncb_v, idx_v, val_v, out_v, sem):
    W = Lseq - KMER + 1
    RPW = B // NW
    GCH = GROW // LANES
    GSZ = W * GROW
    POW_TOP = ALPHA ** (KMER - 1)

    cid = lax.axis_index("c")
    sid = lax.axis_index("s")
    wid = sid * NC + cid
    base = wid * RPW

    pltpu.sync_copy(cand_hbm.at[pl.ds(base, RPW), :], slab)
    pltpu.sync_copy(ncb_hbm, ncb_v)

    zeros16 = jnp.zeros((LANES,), jnp.int32)

    def compute_idx(g):
        def idx_chunk(c, carry):
            rows = (g * GROW + c * LANES) + lax.iota(jnp.int32, LANES)
            cols = {}

            def col(j):
                if j not in cols:
                    cols[j] = plsc.load_gather(
                        slab, [rows, jnp.full((LANES,), j, jnp.int32)])
                return cols[j]

            h = col(0)
            for j in range(1, KMER):
                h = h * ALPHA + col(j)
            idx_v[pl.ds(g * GSZ + c * LANES, LANES)] = h
            for w in range(1, W):
                h = (h - col(w - 1) * POW_TOP) * ALPHA + col(w + KMER - 1)
                idx_v[pl.ds(g * GSZ + w * GROW + c * LANES, LANES)] = h
            return carry

        lax.fori_loop(0, GCH, idx_chunk, None)

    def gather_dma(g):
        return pltpu.make_async_copy(
            table_hbm.at[idx_v.at[pl.ds(g * GSZ, GSZ)]],
            val_v.at[pl.ds(g * GSZ, GSZ)], sem.at[g])

    inv_p = 1.0 / P

    def reduce_group(g):
        def red_chunk(c, carry):
            cnt = val_v[pl.ds(g * GSZ + c * LANES, LANES)]
            for w in range(1, W):
                cnt = cnt + val_v[pl.ds(g * GSZ + w * GROW + c * LANES, LANES)]
            contrib = jnp.minimum(cnt, 1.0) * MOTIF_EFFECT
            s = jnp.zeros((LANES,), jnp.float32)
            for p in range(P):
                x = contrib + ncb_v[p, :]
                s = s + 1.0 / (1.0 + jnp.exp(-x))
            out_v[pl.ds(g * GROW + c * LANES, LANES)] = s * inv_p
            return carry

        lax.fori_loop(0, GCH, red_chunk, None)

    compute_idx(0)
    gather_dma(0).start()
    for g in range(1, GROUPS):
        compute_idx(g)
        gather_dma(g).start()
    for g in range(GROUPS):
        gather_dma(g).wait()
        reduce_group(g)

    pltpu.sync_copy(out_v, out_hbm.at[pl.ds(base, RPW)])


@jax.jit
def kernel(candidates, table, natural_contribs):
    B, Lseq = candidates.shape
    P = natural_contribs.shape[0]
    W = Lseq - KMER + 1
    RPW = B // NW

    cand = candidates.astype(jnp.int32)
    tab = table.reshape(-1).astype(jnp.float32)
    ncb = jnp.broadcast_to(
        natural_contribs.astype(jnp.float32)[:, None], (P, LANES))

    mesh = plsc.VectorSubcoreMesh(
        core_axis_name="c", subcore_axis_name="s",
        num_cores=NC, num_subcores=NS)

    run = pl.kernel(
        functools.partial(_body, B, Lseq, P),
        out_type=jax.ShapeDtypeStruct((B,), jnp.float32),
        mesh=mesh,
        compiler_params=pltpu.CompilerParams(needs_layout_passes=False),
        scratch_types=[
            pltpu.VMEM((RPW, Lseq), jnp.int32),
            pltpu.VMEM((P, LANES), jnp.float32),
            pltpu.VMEM((GROUPS * W * GROW,), jnp.int32),
            pltpu.VMEM((GROUPS * W * GROW,), jnp.float32),
            pltpu.VMEM((RPW,), jnp.float32),
            pltpu.SemaphoreType.DMA((GROUPS,)),
        ],
    )
    return run(cand, tab, ncb)


# speedup vs baseline: 2.3185x; 1.0027x over previous
import functools

import jax
import jax.numpy as jnp
from jax import lax
from jax.experimental import pallas as pl
from jax.experimental.pallas import tpu as pltpu
from jax.experimental.pallas import tpu_sc as plsc

ALPHA = 20
KMER = 5
MOTIF_EFFECT = 2.0
NC, NS, LANES = 2, 16, 16
NW = NC * NS
GROUPS = 4
GROW = 128


def _body(B, Lseq, P, cand_hbm, table_hbm, ncb_hbm, out_hbm,
          slab, ncb_v, encb_v, idx_v, val_v, out_v, sem):
    W = Lseq - KMER + 1
    RPW = B // NW
    GCH = GROW // LANES
    GSZ = W * GROW
    POW_TOP = ALPHA ** (KMER - 1)

    cid = lax.axis_index("c")
    sid = lax.axis_index("s")
    wid = sid * NC + cid
    base = wid * RPW

    pltpu.sync_copy(cand_hbm.at[pl.ds(base, RPW), :], slab)
    pltpu.sync_copy(ncb_hbm, ncb_v)
    for p in range(P):
        encb_v[p, :] = jnp.exp(-ncb_v[p, :])

    zeros16 = jnp.zeros((LANES,), jnp.int32)

    def compute_idx(g):
        def idx_chunk(c, carry):
            rows = (g * GROW + c * LANES) + lax.iota(jnp.int32, LANES)
            cols = {}

            def col(j):
                if j not in cols:
                    cols[j] = plsc.load_gather(
                        slab, [rows, jnp.full((LANES,), j, jnp.int32)])
                return cols[j]

            h = col(0)
            for j in range(1, KMER):
                h = h * ALPHA + col(j)
            idx_v[pl.ds(g * GSZ + c * LANES, LANES)] = h
            for w in range(1, W):
                h = (h - col(w - 1) * POW_TOP) * ALPHA + col(w + KMER - 1)
                idx_v[pl.ds(g * GSZ + w * GROW + c * LANES, LANES)] = h
            return carry

        lax.fori_loop(0, GCH, idx_chunk, None)

    def gather_dma(g):
        return pltpu.make_async_copy(
            table_hbm.at[idx_v.at[pl.ds(g * GSZ, GSZ)]],
            val_v.at[pl.ds(g * GSZ, GSZ)], sem.at[g])

    inv_p = 1.0 / P

    def reduce_group(g):
        def red_chunk(c, carry):
            cnt = val_v[pl.ds(g * GSZ + c * LANES, LANES)]
            for w in range(1, W):
                cnt = cnt + val_v[pl.ds(g * GSZ + w * GROW + c * LANES, LANES)]
            contrib = jnp.minimum(cnt, 1.0) * MOTIF_EFFECT
            t = jnp.exp(-contrib)
            s = jnp.zeros((LANES,), jnp.float32)
            for p in range(P):
                s = s + 1.0 / (1.0 + t * encb_v[p, :])
            out_v[pl.ds(g * GROW + c * LANES, LANES)] = s * inv_p
            return carry

        lax.fori_loop(0, GCH, red_chunk, None)

    compute_idx(0)
    gather_dma(0).start()
    for g in range(1, GROUPS):
        compute_idx(g)
        gather_dma(g).start()
    for g in range(GROUPS):
        gather_dma(g).wait()
        reduce_group(g)

    pltpu.sync_copy(out_v, out_hbm.at[pl.ds(base, RPW)])


@jax.jit
def kernel(candidates, table, natural_contribs):
    B, Lseq = candidates.shape
    P = natural_contribs.shape[0]
    W = Lseq - KMER + 1
    RPW = B // NW

    cand = candidates.astype(jnp.int32)
    tab = table.reshape(-1).astype(jnp.float32)
    ncb = jnp.broadcast_to(
        natural_contribs.astype(jnp.float32)[:, None], (P, LANES))

    mesh = plsc.VectorSubcoreMesh(
        core_axis_name="c", subcore_axis_name="s",
        num_cores=NC, num_subcores=NS)

    run = pl.kernel(
        functools.partial(_body, B, Lseq, P),
        out_type=jax.ShapeDtypeStruct((B,), jnp.float32),
        mesh=mesh,
        compiler_params=pltpu.CompilerParams(needs_layout_passes=False),
        scratch_types=[
            pltpu.VMEM((RPW, Lseq), jnp.int32),
            pltpu.VMEM((P, LANES), jnp.float32),
            pltpu.VMEM((P, LANES), jnp.float32),
            pltpu.VMEM((GROUPS * W * GROW,), jnp.int32),
            pltpu.VMEM((GROUPS * W * GROW,), jnp.float32),
            pltpu.VMEM((RPW,), jnp.float32),
            pltpu.SemaphoreType.DMA((GROUPS,)),
        ],
    )
    return run(cand, tab, ncb)


# speedup vs baseline: 2.3540x; 1.0153x over previous
import functools

import jax
import jax.numpy as jnp
from jax import lax
from jax.experimental import pallas as pl
from jax.experimental.pallas import tpu as pltpu
from jax.experimental.pallas import tpu_sc as plsc

ALPHA = 20
KMER = 5
MOTIF_EFFECT = 2.0
NC, NS, LANES = 2, 16, 16
NW = NC * NS
GROUPS = 8
GROW = 64


def _body(B, Lseq, P, cand_hbm, table_hbm, ncb_hbm, out_hbm,
          slab, ncb_v, encb_v, idx_v, val_v, out_v, sem, ssem):
    W = Lseq - KMER + 1
    RPW = B // NW
    GCH = GROW // LANES
    GSZ = W * GROW
    POW_TOP = ALPHA ** (KMER - 1)

    cid = lax.axis_index("c")
    sid = lax.axis_index("s")
    wid = sid * NC + cid
    base = wid * RPW

    def slab_dma(g):
        return pltpu.make_async_copy(
            cand_hbm.at[pl.ds(base + g * GROW, GROW), :],
            slab.at[pl.ds(g * GROW, GROW), :], ssem.at[g])

    for g in range(GROUPS):
        slab_dma(g).start()
    pltpu.sync_copy(ncb_hbm, ncb_v)
    for p in range(P):
        encb_v[p, :] = jnp.exp(-ncb_v[p, :])

    zeros16 = jnp.zeros((LANES,), jnp.int32)

    def compute_idx(g):
        def idx_chunk(c, carry):
            rows = (g * GROW + c * LANES) + lax.iota(jnp.int32, LANES)
            cols = {}

            def col(j):
                if j not in cols:
                    cols[j] = plsc.load_gather(
                        slab, [rows, jnp.full((LANES,), j, jnp.int32)])
                return cols[j]

            h = col(0)
            for j in range(1, KMER):
                h = h * ALPHA + col(j)
            idx_v[pl.ds(g * GSZ + c * LANES, LANES)] = h
            for w in range(1, W):
                h = (h - col(w - 1) * POW_TOP) * ALPHA + col(w + KMER - 1)
                idx_v[pl.ds(g * GSZ + w * GROW + c * LANES, LANES)] = h
            return carry

        lax.fori_loop(0, GCH, idx_chunk, None)

    def gather_dma(g):
        return pltpu.make_async_copy(
            table_hbm.at[idx_v.at[pl.ds(g * GSZ, GSZ)]],
            val_v.at[pl.ds(g * GSZ, GSZ)], sem.at[g])

    inv_p = 1.0 / P

    def reduce_group(g):
        def red_chunk(c, carry):
            cnt = val_v[pl.ds(g * GSZ + c * LANES, LANES)]
            for w in range(1, W):
                cnt = cnt + val_v[pl.ds(g * GSZ + w * GROW + c * LANES, LANES)]
            contrib = jnp.minimum(cnt, 1.0) * MOTIF_EFFECT
            t = jnp.exp(-contrib)
            s = jnp.zeros((LANES,), jnp.float32)
            for p in range(P):
                s = s + 1.0 / (1.0 + t * encb_v[p, :])
            out_v[pl.ds(g * GROW + c * LANES, LANES)] = s * inv_p
            return carry

        lax.fori_loop(0, GCH, red_chunk, None)

    slab_dma(0).wait()
    compute_idx(0)
    gather_dma(0).start()
    for g in range(1, GROUPS):
        slab_dma(g).wait()
        compute_idx(g)
        gather_dma(g).start()
    for g in range(GROUPS):
        gather_dma(g).wait()
        reduce_group(g)

    pltpu.sync_copy(out_v, out_hbm.at[pl.ds(base, RPW)])


@jax.jit
def kernel(candidates, table, natural_contribs):
    B, Lseq = candidates.shape
    P = natural_contribs.shape[0]
    W = Lseq - KMER + 1
    RPW = B // NW

    cand = candidates.astype(jnp.int32)
    tab = table.reshape(-1).astype(jnp.float32)
    ncb = jnp.broadcast_to(
        natural_contribs.astype(jnp.float32)[:, None], (P, LANES))

    mesh = plsc.VectorSubcoreMesh(
        core_axis_name="c", subcore_axis_name="s",
        num_cores=NC, num_subcores=NS)

    run = pl.kernel(
        functools.partial(_body, B, Lseq, P),
        out_type=jax.ShapeDtypeStruct((B,), jnp.float32),
        mesh=mesh,
        compiler_params=pltpu.CompilerParams(needs_layout_passes=False),
        scratch_types=[
            pltpu.VMEM((RPW, Lseq), jnp.int32),
            pltpu.VMEM((P, LANES), jnp.float32),
            pltpu.VMEM((P, LANES), jnp.float32),
            pltpu.VMEM((GROUPS * W * GROW,), jnp.int32),
            pltpu.VMEM((GROUPS * W * GROW,), jnp.float32),
            pltpu.VMEM((RPW,), jnp.float32),
            pltpu.SemaphoreType.DMA((GROUPS,)),
            pltpu.SemaphoreType.DMA((GROUPS,)),
        ],
    )
    return run(cand, tab, ncb)
